# baseline (speedup 1.0000x reference)
import functools

import jax
import jax.numpy as jnp
from jax import lax
from jax.experimental import pallas as pl
from jax.experimental.pallas import tpu as pltpu
from jax.experimental.pallas import tpu_sc as plsc

UNITS = 128
REPS = 5
N_NODES = 10000
N_EDGES = 320000

NUM_CORES = 2
NUM_SUBCORES = 16
NUM_WORKERS = NUM_CORES * NUM_SUBCORES
CHUNK = 128
N_CHUNKS = 80
EDGES_PER_WORKER = N_CHUNKS * CHUNK
EDGES_PAD = NUM_WORKERS * EDGES_PER_WORKER
ACC_ROWS_PER_SUBCORE = 640
ACC_ROWS = ACC_ROWS_PER_SUBCORE * NUM_SUBCORES
DUMP_ROW = N_NODES


def _sc_segment_sum_body(state_hbm, src_hbm, dst_hbm, zeros_hbm, out_hbm,
                         acc_smem, src_v, dst_v, rows_v, sem):
    cid = lax.axis_index("c")
    sid = lax.axis_index("s")
    wid = cid * NUM_SUBCORES + sid

    pltpu.sync_copy(zeros_hbm, acc_smem.at[pl.ds(sid * ACC_ROWS_PER_SUBCORE,
                                                 ACC_ROWS_PER_SUBCORE)])
    pltpu.sync_copy(src_hbm.at[wid], src_v)
    pltpu.sync_copy(dst_hbm.at[wid], dst_v)
    plsc.subcore_barrier()

    def chunk_step(j, carry):
        pltpu.async_copy(state_hbm.at[src_v.at[pl.ds(j * CHUNK, CHUNK)]],
                         rows_v, sem).wait()
        pltpu.sync_copy(rows_v, acc_smem.at[dst_v.at[j]], add=True)
        return carry

    lax.fori_loop(0, N_CHUNKS, chunk_step, 0)
    plsc.subcore_barrier()

    pltpu.sync_copy(
        acc_smem.at[pl.ds(sid * ACC_ROWS_PER_SUBCORE, ACC_ROWS_PER_SUBCORE)],
        out_hbm.at[cid, pl.ds(sid * ACC_ROWS_PER_SUBCORE, ACC_ROWS_PER_SUBCORE)])


@functools.cache
def _sc_segment_sum():
    return pl.kernel(
        _sc_segment_sum_body,
        out_type=jax.ShapeDtypeStruct((NUM_CORES, ACC_ROWS, UNITS), jnp.float32),
        mesh=plsc.VectorSubcoreMesh(core_axis_name="c", subcore_axis_name="s",
                                    num_cores=NUM_CORES,
                                    num_subcores=NUM_SUBCORES),
        scratch_types=[
            pltpu.VMEM_SHARED((ACC_ROWS, UNITS), jnp.float32),
            pltpu.VMEM((EDGES_PER_WORKER,), jnp.int32),
            pltpu.VMEM((N_CHUNKS, CHUNK), jnp.int32),
            pltpu.VMEM((CHUNK, UNITS), jnp.float32),
            pltpu.SemaphoreType.DMA,
        ],
    )


def _tc_dense_body(parts_ref, w_ref, bl_ref, gk_ref, gb_ref, out_ref):
    x = parts_ref[0] + parts_ref[1]
    h1 = jnp.dot(x, w_ref[...], preferred_element_type=jnp.float32) + bl_ref[...]
    mx = jnp.dot(h1, gk_ref[...], preferred_element_type=jnp.float32) + gb_ref[0:1, :]
    rec = gb_ref[1:2, :]
    z = jax.nn.sigmoid(mx[:, :UNITS] + rec[:, :UNITS])
    r = jax.nn.sigmoid(mx[:, UNITS:2 * UNITS] + rec[:, UNITS:2 * UNITS])
    hh = jnp.tanh(mx[:, 2 * UNITS:] + r * rec[:, 2 * UNITS:])
    res = (1.0 - z) * hh
    sq = jnp.sum(res * res, axis=1, keepdims=True)
    out_ref[...] = res * lax.rsqrt(jnp.maximum(sq, 1e-12))


_TC_BLOCK = 2048


def _tc_dense(parts, w, bl, gk, gb):
    grid = ACC_ROWS // _TC_BLOCK
    return pl.pallas_call(
        _tc_dense_body,
        grid=(grid,),
        in_specs=[
            pl.BlockSpec((NUM_CORES, _TC_BLOCK, UNITS), lambda i: (0, i, 0)),
            pl.BlockSpec((UNITS, UNITS), lambda i: (0, 0)),
            pl.BlockSpec((1, UNITS), lambda i: (0, 0)),
            pl.BlockSpec((UNITS, 3 * UNITS), lambda i: (0, 0)),
            pl.BlockSpec((2, 3 * UNITS), lambda i: (0, 0)),
        ],
        out_specs=pl.BlockSpec((_TC_BLOCK, UNITS), lambda i: (i, 0)),
        out_shape=jax.ShapeDtypeStruct((ACC_ROWS, UNITS), jnp.float32),
    )(parts, w, bl, gk, gb)


def kernel(message, edge_index, W_lin, b_lin, gru_kernel, gru_rec_kernel, gru_bias):
    del gru_rec_kernel
    src = edge_index[0].astype(jnp.int32)
    dst = edge_index[1].astype(jnp.int32)
    pad = EDGES_PAD - N_EDGES
    src2 = jnp.concatenate([src, jnp.zeros((pad,), jnp.int32)]).reshape(
        NUM_WORKERS, EDGES_PER_WORKER)
    dst3 = jnp.concatenate([dst, jnp.full((pad,), DUMP_ROW, jnp.int32)]).reshape(
        NUM_WORKERS, N_CHUNKS, CHUNK)
    zeros = jnp.zeros((ACC_ROWS_PER_SUBCORE, UNITS), jnp.float32)
    bl2 = b_lin.reshape(1, UNITS)

    state = jnp.concatenate(
        [message, jnp.zeros((ACC_ROWS - N_NODES, UNITS), jnp.float32)])
    outs = []
    for _ in range(REPS):
        parts = _sc_segment_sum()(state, src2, dst3, zeros)
        state = _tc_dense(parts, W_lin, bl2, gru_kernel, gru_bias)
        outs.append(state)
    out = jnp.concatenate(outs, axis=-1)[:N_NODES]
    return jnp.reshape(out, (N_NODES, UNITS, REPS))


# speedup vs baseline: 1.0681x; 1.0681x over previous
import functools

import jax
import jax.numpy as jnp
from jax import lax
from jax.experimental import pallas as pl
from jax.experimental.pallas import tpu as pltpu
from jax.experimental.pallas import tpu_sc as plsc

UNITS = 128
REPS = 5
N_NODES = 10000
N_EDGES = 320000

NUM_CORES = 2
NUM_SUBCORES = 16
NUM_WORKERS = NUM_CORES * NUM_SUBCORES
CHUNK = 128
N_CHUNKS = 80
EDGES_PER_WORKER = N_CHUNKS * CHUNK
EDGES_PAD = NUM_WORKERS * EDGES_PER_WORKER
ACC_ROWS_PER_SUBCORE = 640
ACC_ROWS = ACC_ROWS_PER_SUBCORE * NUM_SUBCORES
DUMP_ROW = N_NODES


def _sc_segment_sum_body(state_hbm, src_hbm, dst_hbm, zeros_hbm, out_hbm,
                         acc_smem, src_v, dst_v, rows_v, sem):
    cid = lax.axis_index("c")
    sid = lax.axis_index("s")
    wid = cid * NUM_SUBCORES + sid

    pltpu.sync_copy(zeros_hbm, acc_smem.at[pl.ds(sid * ACC_ROWS_PER_SUBCORE,
                                                 ACC_ROWS_PER_SUBCORE)])
    pltpu.sync_copy(src_hbm.at[wid], src_v)
    pltpu.sync_copy(dst_hbm.at[wid], dst_v)
    plsc.subcore_barrier()

    def chunk_step(j, carry):
        pltpu.async_copy(state_hbm.at[src_v.at[j]], rows_v, sem).wait()
        pltpu.sync_copy(rows_v, acc_smem.at[dst_v.at[j]], add=True)
        return carry

    lax.fori_loop(0, N_CHUNKS, chunk_step, 0)
    plsc.subcore_barrier()

    pltpu.sync_copy(
        acc_smem.at[pl.ds(sid * ACC_ROWS_PER_SUBCORE, ACC_ROWS_PER_SUBCORE)],
        out_hbm.at[cid, pl.ds(sid * ACC_ROWS_PER_SUBCORE, ACC_ROWS_PER_SUBCORE)])


@functools.cache
def _sc_segment_sum():
    return pl.kernel(
        _sc_segment_sum_body,
        out_type=jax.ShapeDtypeStruct((NUM_CORES, ACC_ROWS, UNITS), jnp.float32),
        mesh=plsc.VectorSubcoreMesh(core_axis_name="c", subcore_axis_name="s",
                                    num_cores=NUM_CORES,
                                    num_subcores=NUM_SUBCORES),
        scratch_types=[
            pltpu.VMEM_SHARED((ACC_ROWS, UNITS), jnp.float32),
            pltpu.VMEM((N_CHUNKS, CHUNK), jnp.int32),
            pltpu.VMEM((N_CHUNKS, CHUNK), jnp.int32),
            pltpu.VMEM((CHUNK, UNITS), jnp.float32),
            pltpu.SemaphoreType.DMA,
        ],
    )


def _tc_dense_body(parts_ref, w_ref, bl_ref, gk_ref, gb_ref, out_ref):
    x = parts_ref[0] + parts_ref[1]
    h1 = jnp.dot(x, w_ref[...], preferred_element_type=jnp.float32) + bl_ref[...]
    mx = jnp.dot(h1, gk_ref[...], preferred_element_type=jnp.float32) + gb_ref[0:1, :]
    rec = gb_ref[1:2, :]
    z = jax.nn.sigmoid(mx[:, :UNITS] + rec[:, :UNITS])
    r = jax.nn.sigmoid(mx[:, UNITS:2 * UNITS] + rec[:, UNITS:2 * UNITS])
    hh = jnp.tanh(mx[:, 2 * UNITS:] + r * rec[:, 2 * UNITS:])
    res = (1.0 - z) * hh
    sq = jnp.sum(res * res, axis=1, keepdims=True)
    out_ref[...] = res * lax.rsqrt(jnp.maximum(sq, 1e-12))


_TC_BLOCK = 2048


def _tc_dense(parts, w, bl, gk, gb):
    grid = ACC_ROWS // _TC_BLOCK
    return pl.pallas_call(
        _tc_dense_body,
        grid=(grid,),
        in_specs=[
            pl.BlockSpec((NUM_CORES, _TC_BLOCK, UNITS), lambda i: (0, i, 0)),
            pl.BlockSpec((UNITS, UNITS), lambda i: (0, 0)),
            pl.BlockSpec((1, UNITS), lambda i: (0, 0)),
            pl.BlockSpec((UNITS, 3 * UNITS), lambda i: (0, 0)),
            pl.BlockSpec((2, 3 * UNITS), lambda i: (0, 0)),
        ],
        out_specs=pl.BlockSpec((_TC_BLOCK, UNITS), lambda i: (i, 0)),
        out_shape=jax.ShapeDtypeStruct((ACC_ROWS, UNITS), jnp.float32),
    )(parts, w, bl, gk, gb)


def kernel(message, edge_index, W_lin, b_lin, gru_kernel, gru_rec_kernel, gru_bias):
    del gru_rec_kernel
    src = edge_index[0].astype(jnp.int32)
    dst = edge_index[1].astype(jnp.int32)
    pad = EDGES_PAD - N_EDGES
    src2 = jnp.concatenate([src, jnp.zeros((pad,), jnp.int32)]).reshape(
        NUM_WORKERS, N_CHUNKS, CHUNK)
    dst3 = jnp.concatenate([dst, jnp.full((pad,), DUMP_ROW, jnp.int32)]).reshape(
        NUM_WORKERS, N_CHUNKS, CHUNK)
    zeros = jnp.zeros((ACC_ROWS_PER_SUBCORE, UNITS), jnp.float32)
    bl2 = b_lin.reshape(1, UNITS)

    state = jnp.concatenate(
        [message, jnp.zeros((ACC_ROWS - N_NODES, UNITS), jnp.float32)])
    outs = []
    for _ in range(REPS):
        parts = _sc_segment_sum()(state, src2, dst3, zeros)
        state = _tc_dense(parts, W_lin, bl2, gru_kernel, gru_bias)
        outs.append(state)
    out = jnp.concatenate(outs, axis=-1)[:N_NODES]
    return jnp.reshape(out, (N_NODES, UNITS, REPS))


# speedup vs baseline: 1.2034x; 1.1267x over previous
import functools

import jax
import jax.numpy as jnp
from jax import lax
from jax.experimental import pallas as pl
from jax.experimental.pallas import tpu as pltpu
from jax.experimental.pallas import tpu_sc as plsc

UNITS = 128
REPS = 5
N_NODES = 10000
N_EDGES = 320000

NUM_CORES = 2
NUM_SUBCORES = 16
NUM_WORKERS = NUM_CORES * NUM_SUBCORES
CHUNK = 128
N_CHUNKS = 80
EDGES_PER_WORKER = N_CHUNKS * CHUNK
EDGES_PAD = NUM_WORKERS * EDGES_PER_WORKER
ACC_ROWS_PER_SUBCORE = 640
ACC_ROWS = ACC_ROWS_PER_SUBCORE * NUM_SUBCORES
DUMP_ROW = N_NODES


def _sc_segment_sum_body(state_hbm, src_hbm, dst_hbm, zeros_hbm, out_hbm,
                         acc_smem, src_v, dst_v, rows_v, sem):
    cid = lax.axis_index("c")
    sid = lax.axis_index("s")
    wid = cid * NUM_SUBCORES + sid

    pltpu.sync_copy(zeros_hbm, acc_smem.at[pl.ds(sid * ACC_ROWS_PER_SUBCORE,
                                                 ACC_ROWS_PER_SUBCORE)])
    pltpu.sync_copy(src_hbm.at[wid], src_v)
    pltpu.sync_copy(dst_hbm.at[wid], dst_v)
    plsc.subcore_barrier()

    def chunk_step(j, carry):
        pltpu.async_copy(state_hbm.at[src_v.at[j]], rows_v, sem).wait()
        pltpu.sync_copy(rows_v, acc_smem.at[dst_v.at[j]], add=True)
        return carry

    lax.fori_loop(0, N_CHUNKS, chunk_step, 0)
    plsc.subcore_barrier()

    pltpu.sync_copy(
        acc_smem.at[pl.ds(sid * ACC_ROWS_PER_SUBCORE, ACC_ROWS_PER_SUBCORE)],
        out_hbm.at[cid, pl.ds(sid * ACC_ROWS_PER_SUBCORE, ACC_ROWS_PER_SUBCORE)])


@functools.cache
def _sc_segment_sum():
    return pl.kernel(
        _sc_segment_sum_body,
        out_type=jax.ShapeDtypeStruct((NUM_CORES, ACC_ROWS, UNITS), jnp.float32),
        mesh=plsc.VectorSubcoreMesh(core_axis_name="c", subcore_axis_name="s",
                                    num_cores=NUM_CORES,
                                    num_subcores=NUM_SUBCORES),
        scratch_types=[
            pltpu.VMEM_SHARED((ACC_ROWS, UNITS), jnp.float32),
            pltpu.VMEM((N_CHUNKS, CHUNK), jnp.int32),
            pltpu.VMEM((N_CHUNKS, CHUNK), jnp.int32),
            pltpu.VMEM((CHUNK, UNITS), jnp.float32),
            pltpu.SemaphoreType.DMA,
        ],
    )


def _tc_dense_body(parts_ref, w_ref, bl_ref, gk_ref, gb_ref, out_ref):
    x = parts_ref[0] + parts_ref[1]
    h1 = jnp.dot(x, w_ref[...], preferred_element_type=jnp.float32) + bl_ref[...]
    mx = jnp.dot(h1, gk_ref[...], preferred_element_type=jnp.float32) + gb_ref[0:1, :]
    rec = gb_ref[1:2, :]
    z = jax.nn.sigmoid(mx[:, :UNITS] + rec[:, :UNITS])
    r = jax.nn.sigmoid(mx[:, UNITS:2 * UNITS] + rec[:, UNITS:2 * UNITS])
    hh = jnp.tanh(mx[:, 2 * UNITS:] + r * rec[:, 2 * UNITS:])
    res = (1.0 - z) * hh
    sq = jnp.sum(res * res, axis=1, keepdims=True)
    out_ref[...] = res * lax.rsqrt(jnp.maximum(sq, 1e-12))


_TC_BLOCK = 2048


def _tc_dense(parts, w, bl, gk, gb):
    grid = ACC_ROWS // _TC_BLOCK
    return pl.pallas_call(
        _tc_dense_body,
        grid=(grid,),
        in_specs=[
            pl.BlockSpec((NUM_CORES, _TC_BLOCK, UNITS), lambda i: (0, i, 0)),
            pl.BlockSpec((UNITS, UNITS), lambda i: (0, 0)),
            pl.BlockSpec((1, UNITS), lambda i: (0, 0)),
            pl.BlockSpec((UNITS, 3 * UNITS), lambda i: (0, 0)),
            pl.BlockSpec((2, 3 * UNITS), lambda i: (0, 0)),
        ],
        out_specs=pl.BlockSpec((_TC_BLOCK, UNITS), lambda i: (i, 0)),
        out_shape=jax.ShapeDtypeStruct((ACC_ROWS, UNITS), jnp.float32),
    )(parts, w, bl, gk, gb)


def kernel(message, edge_index, W_lin, b_lin, gru_kernel, gru_rec_kernel, gru_bias):
    del gru_rec_kernel
    src = edge_index[0].astype(jnp.int32)
    dst = edge_index[1].astype(jnp.int32)
    real_per_worker = N_EDGES // NUM_WORKERS
    pad_per_worker = EDGES_PER_WORKER - real_per_worker
    pad_dst = jnp.broadcast_to(
        N_NODES + jnp.arange(pad_per_worker, dtype=jnp.int32),
        (NUM_WORKERS, pad_per_worker))
    src2 = jnp.concatenate(
        [src.reshape(NUM_WORKERS, real_per_worker),
         jnp.zeros((NUM_WORKERS, pad_per_worker), jnp.int32)],
        axis=1).reshape(NUM_WORKERS, N_CHUNKS, CHUNK)
    dst3 = jnp.concatenate(
        [dst.reshape(NUM_WORKERS, real_per_worker), pad_dst],
        axis=1).reshape(NUM_WORKERS, N_CHUNKS, CHUNK)
    zeros = jnp.zeros((ACC_ROWS_PER_SUBCORE, UNITS), jnp.float32)
    bl2 = b_lin.reshape(1, UNITS)

    state = jnp.concatenate(
        [message, jnp.zeros((ACC_ROWS - N_NODES, UNITS), jnp.float32)])
    outs = []
    for _ in range(REPS):
        parts = _sc_segment_sum()(state, src2, dst3, zeros)
        state = _tc_dense(parts, W_lin, bl2, gru_kernel, gru_bias)
        outs.append(state)
    out = jnp.concatenate(outs, axis=-1)[:N_NODES]
    return jnp.reshape(out, (N_NODES, UNITS, REPS))


# speedup vs baseline: 2.7689x; 2.3009x over previous
import functools

import jax
import jax.numpy as jnp
from jax import lax
from jax.experimental import pallas as pl
from jax.experimental.pallas import tpu as pltpu
from jax.experimental.pallas import tpu_sc as plsc

UNITS = 128
REPS = 5
N_NODES = 10000
N_EDGES = 320000

NUM_CORES = 2
NUM_SUBCORES = 16
NUM_WORKERS = NUM_CORES * NUM_SUBCORES
CHUNK = 128
N_CHUNKS = 79
EDGES_PER_WORKER = N_CHUNKS * CHUNK
EDGES_PAD = NUM_WORKERS * EDGES_PER_WORKER
ACC_ROWS_PER_SUBCORE = 640
ACC_ROWS = ACC_ROWS_PER_SUBCORE * NUM_SUBCORES


def _sc_segment_sum_body(state_hbm, src_hbm, dst_hbm, zeros_hbm, out_hbm,
                         acc_smem, src_v, dst_v, rows_v, sem):
    cid = lax.axis_index("c")
    sid = lax.axis_index("s")
    wid = cid * NUM_SUBCORES + sid

    pltpu.sync_copy(zeros_hbm, acc_smem.at[pl.ds(sid * ACC_ROWS_PER_SUBCORE,
                                                 ACC_ROWS_PER_SUBCORE)])
    pltpu.sync_copy(src_hbm.at[wid], src_v)
    pltpu.sync_copy(dst_hbm.at[wid], dst_v)
    plsc.subcore_barrier()

    def chunk_step(j, carry):
        pltpu.async_copy(state_hbm.at[src_v.at[j]], rows_v, sem).wait()
        pltpu.sync_copy(rows_v, acc_smem.at[dst_v.at[j]], add=True)
        return carry

    lax.fori_loop(0, N_CHUNKS, chunk_step, 0)
    plsc.subcore_barrier()

    pltpu.sync_copy(
        acc_smem.at[pl.ds(sid * ACC_ROWS_PER_SUBCORE, ACC_ROWS_PER_SUBCORE)],
        out_hbm.at[cid, pl.ds(sid * ACC_ROWS_PER_SUBCORE, ACC_ROWS_PER_SUBCORE)])


@functools.cache
def _sc_segment_sum():
    return pl.kernel(
        _sc_segment_sum_body,
        out_type=jax.ShapeDtypeStruct((NUM_CORES, ACC_ROWS, UNITS), jnp.float32),
        mesh=plsc.VectorSubcoreMesh(core_axis_name="c", subcore_axis_name="s",
                                    num_cores=NUM_CORES,
                                    num_subcores=NUM_SUBCORES),
        scratch_types=[
            pltpu.VMEM_SHARED((ACC_ROWS, UNITS), jnp.float32),
            pltpu.VMEM((N_CHUNKS, CHUNK), jnp.int32),
            pltpu.VMEM((N_CHUNKS, CHUNK), jnp.int32),
            pltpu.VMEM((CHUNK, UNITS), jnp.float32),
            pltpu.SemaphoreType.DMA,
        ],
    )


def _tc_dense_body(parts_ref, w_ref, bl_ref, gk_ref, gb_ref, out_ref):
    x = parts_ref[0] + parts_ref[1]
    h1 = jnp.dot(x, w_ref[...], preferred_element_type=jnp.float32) + bl_ref[...]
    mx = jnp.dot(h1, gk_ref[...], preferred_element_type=jnp.float32) + gb_ref[0:1, :]
    rec = gb_ref[1:2, :]
    z = jax.nn.sigmoid(mx[:, :UNITS] + rec[:, :UNITS])
    r = jax.nn.sigmoid(mx[:, UNITS:2 * UNITS] + rec[:, UNITS:2 * UNITS])
    hh = jnp.tanh(mx[:, 2 * UNITS:] + r * rec[:, 2 * UNITS:])
    res = (1.0 - z) * hh
    sq = jnp.sum(res * res, axis=1, keepdims=True)
    res = res * lax.rsqrt(jnp.maximum(sq, 1e-12))
    row = (pl.program_id(0) * _TC_BLOCK
           + lax.broadcasted_iota(jnp.int32, (_TC_BLOCK, 1), 0))
    out_ref[...] = jnp.where(row < N_NODES, res, 0.0)


_TC_BLOCK = 2048


def _tc_dense(parts, w, bl, gk, gb):
    grid = ACC_ROWS // _TC_BLOCK
    return pl.pallas_call(
        _tc_dense_body,
        grid=(grid,),
        in_specs=[
            pl.BlockSpec((NUM_CORES, _TC_BLOCK, UNITS), lambda i: (0, i, 0)),
            pl.BlockSpec((UNITS, UNITS), lambda i: (0, 0)),
            pl.BlockSpec((1, UNITS), lambda i: (0, 0)),
            pl.BlockSpec((UNITS, 3 * UNITS), lambda i: (0, 0)),
            pl.BlockSpec((2, 3 * UNITS), lambda i: (0, 0)),
        ],
        out_specs=pl.BlockSpec((_TC_BLOCK, UNITS), lambda i: (i, 0)),
        out_shape=jax.ShapeDtypeStruct((ACC_ROWS, UNITS), jnp.float32),
    )(parts, w, bl, gk, gb)


def kernel(message, edge_index, W_lin, b_lin, gru_kernel, gru_rec_kernel, gru_bias):
    del gru_rec_kernel
    src = edge_index[0].astype(jnp.int32)
    dst = edge_index[1].astype(jnp.int32)
    real_per_worker = N_EDGES // NUM_WORKERS
    pad_per_worker = EDGES_PER_WORKER - real_per_worker
    pad_i = jnp.arange(pad_per_worker, dtype=jnp.int32)
    pad_src = jnp.broadcast_to(N_NODES + pad_i, (NUM_WORKERS, pad_per_worker))
    pad_dst = ((jnp.arange(NUM_WORKERS, dtype=jnp.int32) % NUM_SUBCORES)
               [:, None] * ACC_ROWS_PER_SUBCORE + pad_i[None, :])
    src2 = jnp.concatenate(
        [src.reshape(NUM_WORKERS, real_per_worker), pad_src],
        axis=1).reshape(NUM_WORKERS, N_CHUNKS, CHUNK)
    dst3 = jnp.concatenate(
        [dst.reshape(NUM_WORKERS, real_per_worker), pad_dst],
        axis=1).reshape(NUM_WORKERS, N_CHUNKS, CHUNK)
    zeros = jnp.zeros((ACC_ROWS_PER_SUBCORE, UNITS), jnp.float32)
    bl2 = b_lin.reshape(1, UNITS)

    state = jnp.concatenate(
        [message, jnp.zeros((ACC_ROWS - N_NODES, UNITS), jnp.float32)])
    outs = []
    for _ in range(REPS):
        parts = _sc_segment_sum()(state, src2, dst3, zeros)
        state = _tc_dense(parts, W_lin, bl2, gru_kernel, gru_bias)
        outs.append(state)
    out = jnp.concatenate(outs, axis=-1)[:N_NODES]
    return jnp.reshape(out, (N_NODES, UNITS, REPS))
